# baseline (speedup 1.0000x reference)
import functools

import jax
import jax.numpy as jnp
from jax import lax
from jax.experimental import pallas as pl
from jax.experimental.pallas import tpu as pltpu
from jax.experimental.pallas import tpu_sc as plsc

N = 10000
NPAD = 10240
E = 160000
D_IN = 256
D_H = 512
NCH = 4
CW = 128
NC = 2
NS = 16
EPT = E // NS
B = 80
NPH = NPAD // 2
RPT = NPAD // NS
NB = NPAD // 1024


def _sc_degrees(dst, ev):
    mesh = plsc.VectorSubcoreMesh(core_axis_name="c", subcore_axis_name="s")

    @functools.partial(
        pl.kernel,
        out_type=(jax.ShapeDtypeStruct((NS, NPAD), jnp.float32),
                  jax.ShapeDtypeStruct((NS, NPAD), jnp.float32)),
        mesh=mesh,
        compiler_params=pltpu.CompilerParams(needs_layout_passes=False),
        scratch_types=[
            pltpu.VMEM((EPT,), jnp.int32),
            pltpu.VMEM((EPT,), jnp.float32),
            pltpu.VMEM((NPAD,), jnp.float32),
        ],
    )
    def deg_kernel(dst_hbm, ev_hbm, d1p_hbm, d2p_hbm, dbuf, evbuf, acc):
        cid = lax.axis_index("c")
        sid = lax.axis_index("s")

        def zbody(i, c):
            acc[pl.ds(i * 16, 16)] = jnp.zeros((16,), jnp.float32)
            return c
        lax.fori_loop(0, NPAD // 16, zbody, 0)

        base = sid * EPT
        pltpu.sync_copy(dst_hbm.at[pl.ds(base, EPT)], dbuf)
        lane0 = lax.iota(jnp.int32, 16) == 0

        @pl.when(cid == 0)
        def _():
            pltpu.sync_copy(ev_hbm.at[pl.ds(base, EPT)], evbuf)

            def body(i, c):
                idx = jnp.full((16,), i, jnp.int32)
                d16 = plsc.load_gather(dbuf, [idx])
                v16 = plsc.load_gather(evbuf, [idx])
                plsc.addupdate_scatter(acc, [d16], v16, mask=lane0)
                return c
            lax.fori_loop(0, EPT, body, 0)
            pltpu.sync_copy(acc, d1p_hbm.at[sid])

        @pl.when(cid == 1)
        def _():
            def body(i, c):
                idx = jnp.full((16,), i, jnp.int32)
                d16 = plsc.load_gather(dbuf, [idx])
                plsc.addupdate_scatter(acc, [d16], jnp.ones((16,), jnp.float32),
                                       mask=lane0)
                return c
            lax.fori_loop(0, EPT, body, 0)
            pltpu.sync_copy(acc, d2p_hbm.at[sid])

    return deg_kernel(dst, ev)


def _sc_aggregate_pass(yflat, src3, dst3, ev3, weighted, k, h):
    mesh = plsc.VectorSubcoreMesh(core_axis_name="c", subcore_axis_name="s")
    NBLK = EPT // B
    RPH = NPH // NS

    @functools.partial(
        pl.kernel,
        out_type=jax.ShapeDtypeStruct((NC * NPH, CW), jnp.float32),
        mesh=mesh,
        compiler_params=pltpu.CompilerParams(needs_layout_passes=False),
        scratch_types=[
            pltpu.VMEM((NBLK, B), jnp.int32),
            pltpu.VMEM((NBLK, B), jnp.int32),
            pltpu.VMEM((NBLK, B), jnp.float32),
            pltpu.VMEM((B, CW), jnp.float32),
            pltpu.VMEM((B, CW), jnp.float32),
            pltpu.VMEM_SHARED((NPH + 8, CW), jnp.float32),
            pltpu.SemaphoreType.DMA,
            pltpu.SemaphoreType.DMA,
        ],
    )
    def agg_kernel(y_hbm, src_hbm, dst_hbm, ev_hbm, out_hbm,
                   sadj, didx, evb, rows0, rows1, acc, sem0, sem1):
        cid = lax.axis_index("c")
        sid = lax.axis_index("s")
        chunk = cid * (NCH // NC) + k
        cbase = chunk * NPAD
        pltpu.sync_copy(src_hbm.at[sid], sadj)
        pltpu.sync_copy(dst_hbm.at[sid], didx)
        if weighted:
            pltpu.sync_copy(ev_hbm.at[sid], evb)

        def remap(t, c2):
            r = t // (B // 16)
            sl = pl.ds((t % (B // 16)) * 16, 16)
            sadj[r, sl] = sadj[r, sl] + cbase
            d16 = didx[r, sl]
            if h == 0:
                didx[r, sl] = jnp.where(d16 < NPH, d16, NPH)
            else:
                didx[r, sl] = jnp.where(d16 >= NPH, d16 - NPH, NPH)
            return c2
        lax.fori_loop(0, NBLK * (B // 16), remap, 0)

        bufs = (rows0, rows1)
        sems = (sem0, sem1)

        def start_g(p, j):
            pltpu.async_copy(y_hbm.at[sadj.at[j]], bufs[p], sems[p])

        def wait_g(p):
            pltpu.make_async_copy(y_hbm.at[pl.ds(0, B)], bufs[p], sems[p]).wait()

        def scale(p, j):
            if weighted:
                buf = bufs[p]
                for i in range(B):
                    w = plsc.load_gather(evb.at[j], [jnp.full((16,), i, jnp.int32)])
                    for q in range(CW // 16):
                        sl = pl.ds(q * 16, 16)
                        buf[i, sl] = buf[i, sl] * w

        def process(p, j):
            wait_g(p)
            scale(p, j)
            pltpu.sync_copy(bufs[p], acc.at[didx.at[j]], add=True)

        pltpu.sync_copy(y_hbm.at[pl.ds(cbase + h * NPH + sid * RPH, RPH)],
                        acc.at[pl.ds(sid * RPH, RPH)])
        plsc.subcore_barrier()

        start_g(0, 0)
        start_g(1, 1)

        def body(m, c):
            j0 = 2 * m
            j1 = 2 * m + 1
            process(0, j0)
            start_g(0, j0 + 2)
            process(1, j1)
            start_g(1, j1 + 2)
            return c
        lax.fori_loop(0, (NBLK - 3) // 2, body, 0)
        process(0, NBLK - 3)
        start_g(0, NBLK - 1)
        process(1, NBLK - 2)
        process(0, NBLK - 1)

        plsc.subcore_barrier()
        pltpu.sync_copy(acc.at[pl.ds(sid * RPH, RPH)],
                        out_hbm.at[pl.ds(cid * NPH + sid * RPH, RPH)])

    return agg_kernel(yflat, src3, dst3, ev3)


def _sc_aggregate(yflat, src3, dst3, ev3, weighted):
    parts = {}
    for k in range(NCH // NC):
        for h in range(2):
            out = _sc_aggregate_pass(yflat, src3, dst3, ev3, weighted, k, h)
            for c in range(NC):
                parts[(c * (NCH // NC) + k, h)] = out[c * NPH:(c + 1) * NPH]
    chunks = [jnp.concatenate([parts[(c, 0)], parts[(c, 1)]], axis=0)
              for c in range(NCH)]
    return jnp.concatenate(chunks, axis=0)


def _tc_stage1(xpad, W1, d1p, d2p):
    def body(x_ref, w1_ref, d1p_ref, d2p_ref, y1_ref, dinv_ref):
        xw = jnp.dot(x_ref[...], w1_ref[...], preferred_element_type=jnp.float32)
        dinv1 = lax.rsqrt(jnp.sum(d1p_ref[...], axis=0) + 1.0)
        dinv2 = lax.rsqrt(jnp.sum(d2p_ref[...], axis=0) + 1.0)
        y = xw * dinv1[:, None]
        for c in range(NCH):
            y1_ref[c] = y[:, c * CW:(c + 1) * CW]
        dinv_ref[0] = dinv1
        dinv_ref[1] = dinv2

    return pl.pallas_call(
        body,
        grid=(NB,),
        in_specs=[
            pl.BlockSpec((1024, D_IN), lambda i: (i, 0)),
            pl.BlockSpec((D_IN, D_H), lambda i: (0, 0)),
            pl.BlockSpec((NS, 1024), lambda i: (0, i)),
            pl.BlockSpec((NS, 1024), lambda i: (0, i)),
        ],
        out_specs=[
            pl.BlockSpec((NCH, 1024, CW), lambda i: (0, i, 0)),
            pl.BlockSpec((2, 1024), lambda i: (0, i)),
        ],
        out_shape=[
            jax.ShapeDtypeStruct((NCH, NPAD, CW), jnp.float32),
            jax.ShapeDtypeStruct((2, NPAD), jnp.float32),
        ],
    )(xpad, W1, d1p, d2p)


def _tc_stage2(g1, dinvs, b1, W2):
    def body(g1_ref, dinv_ref, b1_ref, w2_ref, y2_ref):
        s = jnp.concatenate([g1_ref[c] for c in range(NCH)], axis=1)
        h = jnp.maximum(s * dinv_ref[0][:, None] + b1_ref[...][None, :], 0.0)
        hw = jnp.dot(h, w2_ref[...], preferred_element_type=jnp.float32)
        y2 = hw * dinv_ref[1][:, None]
        for c in range(NCH):
            y2_ref[c] = y2[:, c * CW:(c + 1) * CW]

    return pl.pallas_call(
        body,
        grid=(NB,),
        in_specs=[
            pl.BlockSpec((NCH, 1024, CW), lambda i: (0, i, 0)),
            pl.BlockSpec((2, 1024), lambda i: (0, i)),
            pl.BlockSpec((D_H,), lambda i: (0,)),
            pl.BlockSpec((D_H, D_H), lambda i: (0, 0)),
        ],
        out_specs=pl.BlockSpec((NCH, 1024, CW), lambda i: (0, i, 0)),
        out_shape=jax.ShapeDtypeStruct((NCH, NPAD, CW), jnp.float32),
    )(g1, dinvs, b1, W2)


def _tc_stage3(g2, dinvs, b2, Wf, bf):
    def body(g2_ref, dinv_ref, b2_ref, wf_ref, bf_ref, out_ref, acc_ref):
        i = pl.program_id(0)
        s = jnp.concatenate([g2_ref[c] for c in range(NCH)], axis=1)
        h2 = jnp.maximum(s * dinv_ref[1][:, None] + b2_ref[...][None, :], 0.0)
        row = lax.broadcasted_iota(jnp.int32, (1024, 1), 0) + i * 1024
        h2 = jnp.where(row < N, h2, 0.0)
        psum = jnp.sum(h2, axis=0, keepdims=True)

        @pl.when(i == 0)
        def _():
            acc_ref[...] = psum

        @pl.when(i > 0)
        def _():
            acc_ref[...] += psum

        @pl.when(i == NB - 1)
        def _():
            pooled = acc_ref[...] * (1.0 / N)
            t = jnp.dot(pooled, wf_ref[...], preferred_element_type=jnp.float32)
            t = jnp.maximum(t + bf_ref[...][None, :], 0.0)
            m = jnp.max(t, axis=-1, keepdims=True)
            e = jnp.exp(t - m)
            out_ref[...] = e / jnp.sum(e, axis=-1, keepdims=True)

    return pl.pallas_call(
        body,
        grid=(NB,),
        in_specs=[
            pl.BlockSpec((NCH, 1024, CW), lambda i: (0, i, 0)),
            pl.BlockSpec((2, 1024), lambda i: (0, i)),
            pl.BlockSpec((D_H,), lambda i: (0,)),
            pl.BlockSpec((D_H, D_H), lambda i: (0, 0)),
            pl.BlockSpec((D_H,), lambda i: (0,)),
        ],
        out_specs=pl.BlockSpec((1, D_H), lambda i: (0, 0)),
        out_shape=jax.ShapeDtypeStruct((1, D_H), jnp.float32),
        scratch_shapes=[pltpu.VMEM((1, D_H), jnp.float32)],
    )(g2, dinvs, b2, Wf, bf)


def kernel(x, edge_index, edge_values, batch, W1, b1, W2, b2, Wf, bf):
    src = edge_index[0]
    dst = edge_index[1]
    xpad = jnp.pad(x, ((0, NPAD - N), (0, 0)))
    nblk = EPT // B
    src3 = src.reshape(NS, nblk, B)
    dst3 = dst.reshape(NS, nblk, B)
    ev3 = edge_values.reshape(NS, nblk, B)

    d1p, d2p = _sc_degrees(dst, edge_values)
    y1, dinvs = _tc_stage1(xpad, W1, d1p, d2p)
    g1 = _sc_aggregate(y1.reshape(NCH * NPAD, CW), src3, dst3, ev3,
                       weighted=True)
    y2 = _tc_stage2(g1.reshape(NCH, NPAD, CW), dinvs, b1, W2)
    g2 = _sc_aggregate(y2.reshape(NCH * NPAD, CW), src3, dst3, ev3,
                       weighted=False)
    return _tc_stage3(g2.reshape(NCH, NPAD, CW), dinvs, b2, Wf, bf)


# speedup vs baseline: 1.4436x; 1.4436x over previous
import functools

import jax
import jax.numpy as jnp
from jax import lax
from jax.experimental import pallas as pl
from jax.experimental.pallas import tpu as pltpu
from jax.experimental.pallas import tpu_sc as plsc

N = 10000
NPAD = 10240
E = 160000
D_IN = 256
D_H = 512
NCH = 4
CW = 128
NC = 2
NS = 16
EPT = E // NS
B = 80
NPH = NPAD // 2
RPT = NPAD // NS
NB = NPAD // 1024


def _sc_degrees(dst, ev):
    mesh = plsc.VectorSubcoreMesh(core_axis_name="c", subcore_axis_name="s")

    @functools.partial(
        pl.kernel,
        out_type=(jax.ShapeDtypeStruct((NS, NPAD), jnp.float32),
                  jax.ShapeDtypeStruct((NS, NPAD), jnp.float32)),
        mesh=mesh,
        compiler_params=pltpu.CompilerParams(needs_layout_passes=False),
        scratch_types=[
            pltpu.VMEM((EPT,), jnp.int32),
            pltpu.VMEM((EPT,), jnp.float32),
            pltpu.VMEM((NPAD,), jnp.float32),
        ],
    )
    def deg_kernel(dst_hbm, ev_hbm, d1p_hbm, d2p_hbm, dbuf, evbuf, acc):
        cid = lax.axis_index("c")
        sid = lax.axis_index("s")

        def zbody(i, c):
            acc[pl.ds(i * 16, 16)] = jnp.zeros((16,), jnp.float32)
            return c
        lax.fori_loop(0, NPAD // 16, zbody, 0)

        base = sid * EPT
        pltpu.sync_copy(dst_hbm.at[pl.ds(base, EPT)], dbuf)
        lane0 = lax.iota(jnp.int32, 16) == 0

        @pl.when(cid == 0)
        def _():
            pltpu.sync_copy(ev_hbm.at[pl.ds(base, EPT)], evbuf)

            def body(i, c):
                idx = jnp.full((16,), i, jnp.int32)
                d16 = plsc.load_gather(dbuf, [idx])
                v16 = plsc.load_gather(evbuf, [idx])
                plsc.addupdate_scatter(acc, [d16], v16, mask=lane0)
                return c
            lax.fori_loop(0, EPT, body, 0)
            pltpu.sync_copy(acc, d1p_hbm.at[sid])

        @pl.when(cid == 1)
        def _():
            def body(i, c):
                idx = jnp.full((16,), i, jnp.int32)
                d16 = plsc.load_gather(dbuf, [idx])
                plsc.addupdate_scatter(acc, [d16], jnp.ones((16,), jnp.float32),
                                       mask=lane0)
                return c
            lax.fori_loop(0, EPT, body, 0)
            pltpu.sync_copy(acc, d2p_hbm.at[sid])

    return deg_kernel(dst, ev)


def _sc_aggregate_pass(yflat, edata, weighted, k):
    mesh = plsc.VectorSubcoreMesh(core_axis_name="c", subcore_axis_name="s")
    NBLK = EPT // B
    RPH = NPAD // NS

    @functools.partial(
        pl.kernel,
        out_type=jax.ShapeDtypeStruct((NC * NPAD, CW), jnp.float32),
        mesh=mesh,
        compiler_params=pltpu.CompilerParams(needs_layout_passes=False),
        scratch_types=[
            pltpu.VMEM((4, B), jnp.int32),
            pltpu.VMEM((4, B), jnp.int32),
            pltpu.VMEM((B, CW), jnp.float32),
            pltpu.VMEM((B, CW), jnp.float32),
            pltpu.VMEM_SHARED((NPAD, CW), jnp.float32),
            pltpu.SemaphoreType.DMA,
            pltpu.SemaphoreType.DMA,
        ],
    )
    def agg_kernel(y_hbm, edata_hbm, out_hbm,
                   ib0, ib1, rows0, rows1, acc, sem0, sem1):
        cid = lax.axis_index("c")
        sid = lax.axis_index("s")
        chunk = cid * (NCH // NC) + k
        cbase = chunk * NPAD

        bufs = (rows0, rows1)
        ibs = (ib0, ib1)
        sems = (sem0, sem1)

        def load_start_g(p, j):
            ib = ibs[p]
            pltpu.sync_copy(edata_hbm.at[sid, j], ib)
            for u in range(B // 16):
                sl = pl.ds(u * 16, 16)
                ib[3, sl] = ib[0, sl] + cbase
            pltpu.async_copy(y_hbm.at[ib.at[3]], bufs[p], sems[p])

        def wait_g(p):
            pltpu.make_async_copy(y_hbm.at[pl.ds(0, B)], bufs[p], sems[p]).wait()

        def scale(p):
            if weighted:
                buf = bufs[p]
                ev = ibs[p].at[2]
                for i in range(B):
                    w = plsc.bitcast(
                        plsc.load_gather(ev, [jnp.full((16,), i, jnp.int32)]),
                        jnp.float32)
                    for q in range(CW // 16):
                        sl = pl.ds(q * 16, 16)
                        buf[i, sl] = buf[i, sl] * w

        def process(p):
            wait_g(p)
            scale(p)
            pltpu.sync_copy(bufs[p], acc.at[ibs[p].at[1]], add=True)

        pltpu.sync_copy(y_hbm.at[pl.ds(cbase + sid * RPH, RPH)],
                        acc.at[pl.ds(sid * RPH, RPH)])
        plsc.subcore_barrier()

        load_start_g(0, 0)
        load_start_g(1, 1)

        def body(m, c):
            j0 = 2 * m
            j1 = 2 * m + 1
            process(0)
            load_start_g(0, j0 + 2)
            process(1)
            load_start_g(1, j1 + 2)
            return c
        lax.fori_loop(0, (NBLK - 3) // 2, body, 0)
        process(0)
        load_start_g(0, NBLK - 1)
        process(1)
        process(0)

        plsc.subcore_barrier()
        pltpu.sync_copy(acc.at[pl.ds(sid * RPH, RPH)],
                        out_hbm.at[pl.ds(cid * NPAD + sid * RPH, RPH)])

    return agg_kernel(yflat, edata)


def _sc_aggregate(yflat, edata, weighted):
    parts = {}
    for k in range(NCH // NC):
        out = _sc_aggregate_pass(yflat, edata, weighted, k)
        for c in range(NC):
            parts[c * (NCH // NC) + k] = out[c * NPAD:(c + 1) * NPAD]
    return jnp.concatenate([parts[c] for c in range(NCH)], axis=0)


def _tc_stage1(xpad, W1, d1p, d2p):
    def body(x_ref, w1_ref, d1p_ref, d2p_ref, y1_ref, dinv_ref):
        xw = jnp.dot(x_ref[...], w1_ref[...], preferred_element_type=jnp.float32)
        dinv1 = lax.rsqrt(jnp.sum(d1p_ref[...], axis=0) + 1.0)
        dinv2 = lax.rsqrt(jnp.sum(d2p_ref[...], axis=0) + 1.0)
        y = xw * dinv1[:, None]
        for c in range(NCH):
            y1_ref[c] = y[:, c * CW:(c + 1) * CW]
        dinv_ref[0] = dinv1
        dinv_ref[1] = dinv2

    return pl.pallas_call(
        body,
        grid=(NB,),
        in_specs=[
            pl.BlockSpec((1024, D_IN), lambda i: (i, 0)),
            pl.BlockSpec((D_IN, D_H), lambda i: (0, 0)),
            pl.BlockSpec((NS, 1024), lambda i: (0, i)),
            pl.BlockSpec((NS, 1024), lambda i: (0, i)),
        ],
        out_specs=[
            pl.BlockSpec((NCH, 1024, CW), lambda i: (0, i, 0)),
            pl.BlockSpec((2, 1024), lambda i: (0, i)),
        ],
        out_shape=[
            jax.ShapeDtypeStruct((NCH, NPAD, CW), jnp.float32),
            jax.ShapeDtypeStruct((2, NPAD), jnp.float32),
        ],
    )(xpad, W1, d1p, d2p)


def _tc_stage2(g1, dinvs, b1, W2):
    def body(g1_ref, dinv_ref, b1_ref, w2_ref, y2_ref):
        s = jnp.concatenate([g1_ref[c] for c in range(NCH)], axis=1)
        h = jnp.maximum(s * dinv_ref[0][:, None] + b1_ref[...][None, :], 0.0)
        hw = jnp.dot(h, w2_ref[...], preferred_element_type=jnp.float32)
        y2 = hw * dinv_ref[1][:, None]
        for c in range(NCH):
            y2_ref[c] = y2[:, c * CW:(c + 1) * CW]

    return pl.pallas_call(
        body,
        grid=(NB,),
        in_specs=[
            pl.BlockSpec((NCH, 1024, CW), lambda i: (0, i, 0)),
            pl.BlockSpec((2, 1024), lambda i: (0, i)),
            pl.BlockSpec((D_H,), lambda i: (0,)),
            pl.BlockSpec((D_H, D_H), lambda i: (0, 0)),
        ],
        out_specs=pl.BlockSpec((NCH, 1024, CW), lambda i: (0, i, 0)),
        out_shape=jax.ShapeDtypeStruct((NCH, NPAD, CW), jnp.float32),
    )(g1, dinvs, b1, W2)


def _tc_stage3(g2, dinvs, b2, Wf, bf):
    def body(g2_ref, dinv_ref, b2_ref, wf_ref, bf_ref, out_ref, acc_ref):
        i = pl.program_id(0)
        s = jnp.concatenate([g2_ref[c] for c in range(NCH)], axis=1)
        h2 = jnp.maximum(s * dinv_ref[1][:, None] + b2_ref[...][None, :], 0.0)
        row = lax.broadcasted_iota(jnp.int32, (1024, 1), 0) + i * 1024
        h2 = jnp.where(row < N, h2, 0.0)
        psum = jnp.sum(h2, axis=0, keepdims=True)

        @pl.when(i == 0)
        def _():
            acc_ref[...] = psum

        @pl.when(i > 0)
        def _():
            acc_ref[...] += psum

        @pl.when(i == NB - 1)
        def _():
            pooled = acc_ref[...] * (1.0 / N)
            t = jnp.dot(pooled, wf_ref[...], preferred_element_type=jnp.float32)
            t = jnp.maximum(t + bf_ref[...][None, :], 0.0)
            m = jnp.max(t, axis=-1, keepdims=True)
            e = jnp.exp(t - m)
            out_ref[...] = e / jnp.sum(e, axis=-1, keepdims=True)

    return pl.pallas_call(
        body,
        grid=(NB,),
        in_specs=[
            pl.BlockSpec((NCH, 1024, CW), lambda i: (0, i, 0)),
            pl.BlockSpec((2, 1024), lambda i: (0, i)),
            pl.BlockSpec((D_H,), lambda i: (0,)),
            pl.BlockSpec((D_H, D_H), lambda i: (0, 0)),
            pl.BlockSpec((D_H,), lambda i: (0,)),
        ],
        out_specs=pl.BlockSpec((1, D_H), lambda i: (0, 0)),
        out_shape=jax.ShapeDtypeStruct((1, D_H), jnp.float32),
        scratch_shapes=[pltpu.VMEM((1, D_H), jnp.float32)],
    )(g2, dinvs, b2, Wf, bf)


def kernel(x, edge_index, edge_values, batch, W1, b1, W2, b2, Wf, bf):
    src = edge_index[0]
    dst = edge_index[1]
    xpad = jnp.pad(x, ((0, NPAD - N), (0, 0)))
    nblk = EPT // B
    evbits = lax.bitcast_convert_type(edge_values, jnp.int32)
    edata = jnp.stack([src.reshape(NS, nblk, B), dst.reshape(NS, nblk, B),
                       evbits.reshape(NS, nblk, B),
                       jnp.zeros((NS, nblk, B), jnp.int32)], axis=2)

    d1p, d2p = _sc_degrees(dst, edge_values)
    y1, dinvs = _tc_stage1(xpad, W1, d1p, d2p)
    g1 = _sc_aggregate(y1.reshape(NCH * NPAD, CW), edata, weighted=True)
    y2 = _tc_stage2(g1.reshape(NCH, NPAD, CW), dinvs, b1, W2)
    g2 = _sc_aggregate(y2.reshape(NCH * NPAD, CW), edata, weighted=False)
    return _tc_stage3(g2.reshape(NCH, NPAD, CW), dinvs, b2, Wf, bf)


# speedup vs baseline: 2.1715x; 1.5042x over previous
import functools

import jax
import jax.numpy as jnp
from jax import lax
from jax.experimental import pallas as pl
from jax.experimental.pallas import tpu as pltpu
from jax.experimental.pallas import tpu_sc as plsc

N = 10000
NPAD = 10240
E = 160000
D_IN = 256
D_H = 512
NCH = 4
CW = 128
NC = 2
NS = 16
EPT = E // NS
B = 80
NPH = NPAD // 2
RPT = NPAD // NS
NB = NPAD // 1024


def _sc_degrees(dst, ev):
    mesh = plsc.VectorSubcoreMesh(core_axis_name="c", subcore_axis_name="s")

    @functools.partial(
        pl.kernel,
        out_type=(jax.ShapeDtypeStruct((NS, NPAD), jnp.float32),
                  jax.ShapeDtypeStruct((NS, NPAD), jnp.float32)),
        mesh=mesh,
        compiler_params=pltpu.CompilerParams(needs_layout_passes=False),
        scratch_types=[
            pltpu.VMEM((EPT,), jnp.int32),
            pltpu.VMEM((EPT,), jnp.float32),
            pltpu.VMEM((NPAD,), jnp.float32),
        ],
    )
    def deg_kernel(dst_hbm, ev_hbm, d1p_hbm, d2p_hbm, dbuf, evbuf, acc):
        cid = lax.axis_index("c")
        sid = lax.axis_index("s")

        def zbody(i, c):
            acc[pl.ds(i * 16, 16)] = jnp.zeros((16,), jnp.float32)
            return c
        lax.fori_loop(0, NPAD // 16, zbody, 0)

        base = sid * EPT
        pltpu.sync_copy(dst_hbm.at[pl.ds(base, EPT)], dbuf)
        lane0 = lax.iota(jnp.int32, 16) == 0

        @pl.when(cid == 0)
        def _():
            pltpu.sync_copy(ev_hbm.at[pl.ds(base, EPT)], evbuf)

            def body(i, c):
                idx = jnp.full((16,), i, jnp.int32)
                d16 = plsc.load_gather(dbuf, [idx])
                v16 = plsc.load_gather(evbuf, [idx])
                plsc.addupdate_scatter(acc, [d16], v16, mask=lane0)
                return c
            lax.fori_loop(0, EPT, body, 0)
            pltpu.sync_copy(acc, d1p_hbm.at[sid])

        @pl.when(cid == 1)
        def _():
            def body(i, c):
                idx = jnp.full((16,), i, jnp.int32)
                d16 = plsc.load_gather(dbuf, [idx])
                plsc.addupdate_scatter(acc, [d16], jnp.ones((16,), jnp.float32),
                                       mask=lane0)
                return c
            lax.fori_loop(0, EPT, body, 0)
            pltpu.sync_copy(acc, d2p_hbm.at[sid])

    return deg_kernel(dst, ev)


def _sc_aggregate_pass(yflat, edata, weighted, k):
    mesh = plsc.VectorSubcoreMesh(core_axis_name="c", subcore_axis_name="s")
    NBLK = EPT // B
    RPH = NPAD // NS

    @functools.partial(
        pl.kernel,
        out_type=jax.ShapeDtypeStruct((NC * NPAD, CW), jnp.float32),
        mesh=mesh,
        compiler_params=pltpu.CompilerParams(needs_layout_passes=False),
        scratch_types=[
            pltpu.VMEM((4, B), jnp.int32),
            pltpu.VMEM((4, B), jnp.int32),
            pltpu.VMEM((B, CW), jnp.float32),
            pltpu.VMEM((B, CW), jnp.float32),
            pltpu.VMEM_SHARED((NPAD, CW), jnp.float32),
            pltpu.SemaphoreType.DMA,
            pltpu.SemaphoreType.DMA,
        ],
    )
    def agg_kernel(y_hbm, edata_hbm, out_hbm,
                   ib0, ib1, rows0, rows1, acc, sem0, sem1):
        cid = lax.axis_index("c")
        sid = lax.axis_index("s")
        chunk = cid * (NCH // NC) + k
        cbase = chunk * NPAD

        bufs = (rows0, rows1)
        ibs = (ib0, ib1)
        sems = (sem0, sem1)

        def load_start_g(p, j):
            ib = ibs[p]
            pltpu.sync_copy(edata_hbm.at[sid, j], ib)
            for u in range(B // 16):
                sl = pl.ds(u * 16, 16)
                ib[3, sl] = ib[0, sl] + cbase
            pltpu.async_copy(y_hbm.at[ib.at[3]], bufs[p], sems[p])

        def wait_g(p):
            pltpu.make_async_copy(y_hbm.at[pl.ds(0, B)], bufs[p], sems[p]).wait()

        def scale(p):
            if weighted:
                buf = bufs[p]
                ev = ibs[p].at[2]

                @plsc.parallel_loop(0, B, 1, unroll=8)
                def _(i):
                    w = plsc.bitcast(
                        plsc.load_gather(ev, [jnp.full((16,), i, jnp.int32)]),
                        jnp.float32)
                    for q in range(CW // 16):
                        sl = pl.ds(q * 16, 16)
                        buf[i, sl] = buf[i, sl] * w

        def process(p):
            wait_g(p)
            scale(p)
            pltpu.sync_copy(bufs[p], acc.at[ibs[p].at[1]], add=True)

        pltpu.sync_copy(y_hbm.at[pl.ds(cbase + sid * RPH, RPH)],
                        acc.at[pl.ds(sid * RPH, RPH)])
        plsc.subcore_barrier()

        load_start_g(0, 0)
        load_start_g(1, 1)

        def body(m, c):
            j0 = 2 * m
            j1 = 2 * m + 1
            process(0)
            load_start_g(0, j0 + 2)
            process(1)
            load_start_g(1, j1 + 2)
            return c
        lax.fori_loop(0, (NBLK - 3) // 2, body, 0)
        process(0)
        load_start_g(0, NBLK - 1)
        process(1)
        process(0)

        plsc.subcore_barrier()
        pltpu.sync_copy(acc.at[pl.ds(sid * RPH, RPH)],
                        out_hbm.at[pl.ds(cid * NPAD + sid * RPH, RPH)])

    return agg_kernel(yflat, edata)


def _sc_aggregate(yflat, edata, weighted):
    parts = {}
    for k in range(NCH // NC):
        out = _sc_aggregate_pass(yflat, edata, weighted, k)
        for c in range(NC):
            parts[c * (NCH // NC) + k] = out[c * NPAD:(c + 1) * NPAD]
    return jnp.concatenate([parts[c] for c in range(NCH)], axis=0)


def _tc_stage1(xpad, W1, d1p, d2p):
    def body(x_ref, w1_ref, d1p_ref, d2p_ref, y1_ref, dinv_ref):
        xw = jnp.dot(x_ref[...], w1_ref[...], preferred_element_type=jnp.float32)
        dinv1 = lax.rsqrt(jnp.sum(d1p_ref[...], axis=0) + 1.0)
        dinv2 = lax.rsqrt(jnp.sum(d2p_ref[...], axis=0) + 1.0)
        y = xw * dinv1[:, None]
        for c in range(NCH):
            y1_ref[c] = y[:, c * CW:(c + 1) * CW]
        dinv_ref[0] = dinv1
        dinv_ref[1] = dinv2

    return pl.pallas_call(
        body,
        grid=(NB,),
        in_specs=[
            pl.BlockSpec((1024, D_IN), lambda i: (i, 0)),
            pl.BlockSpec((D_IN, D_H), lambda i: (0, 0)),
            pl.BlockSpec((NS, 1024), lambda i: (0, i)),
            pl.BlockSpec((NS, 1024), lambda i: (0, i)),
        ],
        out_specs=[
            pl.BlockSpec((NCH, 1024, CW), lambda i: (0, i, 0)),
            pl.BlockSpec((2, 1024), lambda i: (0, i)),
        ],
        out_shape=[
            jax.ShapeDtypeStruct((NCH, NPAD, CW), jnp.float32),
            jax.ShapeDtypeStruct((2, NPAD), jnp.float32),
        ],
    )(xpad, W1, d1p, d2p)


def _tc_stage2(g1, dinvs, b1, W2):
    def body(g1_ref, dinv_ref, b1_ref, w2_ref, y2_ref):
        s = jnp.concatenate([g1_ref[c] for c in range(NCH)], axis=1)
        h = jnp.maximum(s * dinv_ref[0][:, None] + b1_ref[...][None, :], 0.0)
        hw = jnp.dot(h, w2_ref[...], preferred_element_type=jnp.float32)
        y2 = hw * dinv_ref[1][:, None]
        for c in range(NCH):
            y2_ref[c] = y2[:, c * CW:(c + 1) * CW]

    return pl.pallas_call(
        body,
        grid=(NB,),
        in_specs=[
            pl.BlockSpec((NCH, 1024, CW), lambda i: (0, i, 0)),
            pl.BlockSpec((2, 1024), lambda i: (0, i)),
            pl.BlockSpec((D_H,), lambda i: (0,)),
            pl.BlockSpec((D_H, D_H), lambda i: (0, 0)),
        ],
        out_specs=pl.BlockSpec((NCH, 1024, CW), lambda i: (0, i, 0)),
        out_shape=jax.ShapeDtypeStruct((NCH, NPAD, CW), jnp.float32),
    )(g1, dinvs, b1, W2)


def _tc_stage3(g2, dinvs, b2, Wf, bf):
    def body(g2_ref, dinv_ref, b2_ref, wf_ref, bf_ref, out_ref, acc_ref):
        i = pl.program_id(0)
        s = jnp.concatenate([g2_ref[c] for c in range(NCH)], axis=1)
        h2 = jnp.maximum(s * dinv_ref[1][:, None] + b2_ref[...][None, :], 0.0)
        row = lax.broadcasted_iota(jnp.int32, (1024, 1), 0) + i * 1024
        h2 = jnp.where(row < N, h2, 0.0)
        psum = jnp.sum(h2, axis=0, keepdims=True)

        @pl.when(i == 0)
        def _():
            acc_ref[...] = psum

        @pl.when(i > 0)
        def _():
            acc_ref[...] += psum

        @pl.when(i == NB - 1)
        def _():
            pooled = acc_ref[...] * (1.0 / N)
            t = jnp.dot(pooled, wf_ref[...], preferred_element_type=jnp.float32)
            t = jnp.maximum(t + bf_ref[...][None, :], 0.0)
            m = jnp.max(t, axis=-1, keepdims=True)
            e = jnp.exp(t - m)
            out_ref[...] = e / jnp.sum(e, axis=-1, keepdims=True)

    return pl.pallas_call(
        body,
        grid=(NB,),
        in_specs=[
            pl.BlockSpec((NCH, 1024, CW), lambda i: (0, i, 0)),
            pl.BlockSpec((2, 1024), lambda i: (0, i)),
            pl.BlockSpec((D_H,), lambda i: (0,)),
            pl.BlockSpec((D_H, D_H), lambda i: (0, 0)),
            pl.BlockSpec((D_H,), lambda i: (0,)),
        ],
        out_specs=pl.BlockSpec((1, D_H), lambda i: (0, 0)),
        out_shape=jax.ShapeDtypeStruct((1, D_H), jnp.float32),
        scratch_shapes=[pltpu.VMEM((1, D_H), jnp.float32)],
    )(g2, dinvs, b2, Wf, bf)


def kernel(x, edge_index, edge_values, batch, W1, b1, W2, b2, Wf, bf):
    src = edge_index[0]
    dst = edge_index[1]
    xpad = jnp.pad(x, ((0, NPAD - N), (0, 0)))
    nblk = EPT // B
    evbits = lax.bitcast_convert_type(edge_values, jnp.int32)
    edata = jnp.stack([src.reshape(NS, nblk, B), dst.reshape(NS, nblk, B),
                       evbits.reshape(NS, nblk, B),
                       jnp.zeros((NS, nblk, B), jnp.int32)], axis=2)

    d1p, d2p = _sc_degrees(dst, edge_values)
    y1, dinvs = _tc_stage1(xpad, W1, d1p, d2p)
    g1 = _sc_aggregate(y1.reshape(NCH * NPAD, CW), edata, weighted=True)
    y2 = _tc_stage2(g1.reshape(NCH, NPAD, CW), dinvs, b1, W2)
    g2 = _sc_aggregate(y2.reshape(NCH * NPAD, CW), edata, weighted=False)
    return _tc_stage3(g2.reshape(NCH, NPAD, CW), dinvs, b2, Wf, bf)


# speedup vs baseline: 2.7152x; 1.2504x over previous
import functools

import jax
import jax.numpy as jnp
from jax import lax
from jax.experimental import pallas as pl
from jax.experimental.pallas import tpu as pltpu
from jax.experimental.pallas import tpu_sc as plsc

N = 10000
NPAD = 10240
E = 160000
D_IN = 256
D_H = 512
NCH = 4
CW = 128
NC = 2
NS = 16
EPT = E // NS
B = 80
NPH = NPAD // 2
RPT = NPAD // NS
NB = NPAD // 1024


def _sc_degrees(dst, ev):
    mesh = plsc.VectorSubcoreMesh(core_axis_name="c", subcore_axis_name="s")

    @functools.partial(
        pl.kernel,
        out_type=(jax.ShapeDtypeStruct((NS, NPAD), jnp.float32),
                  jax.ShapeDtypeStruct((NS, NPAD), jnp.float32)),
        mesh=mesh,
        compiler_params=pltpu.CompilerParams(needs_layout_passes=False),
        scratch_types=[
            pltpu.VMEM((EPT,), jnp.int32),
            pltpu.VMEM((EPT,), jnp.float32),
            pltpu.VMEM((NPAD,), jnp.float32),
        ],
    )
    def deg_kernel(dst_hbm, ev_hbm, d1p_hbm, d2p_hbm, dbuf, evbuf, acc):
        cid = lax.axis_index("c")
        sid = lax.axis_index("s")

        def zbody(i, c):
            acc[pl.ds(i * 16, 16)] = jnp.zeros((16,), jnp.float32)
            return c
        lax.fori_loop(0, NPAD // 16, zbody, 0)

        base = sid * EPT
        pltpu.sync_copy(dst_hbm.at[pl.ds(base, EPT)], dbuf)
        lane0 = lax.iota(jnp.int32, 16) == 0

        @pl.when(cid == 0)
        def _():
            pltpu.sync_copy(ev_hbm.at[pl.ds(base, EPT)], evbuf)

            def body(i, c):
                idx = jnp.full((16,), i, jnp.int32)
                d16 = plsc.load_gather(dbuf, [idx])
                v16 = plsc.load_gather(evbuf, [idx])
                plsc.addupdate_scatter(acc, [d16], v16, mask=lane0)
                return c
            lax.fori_loop(0, EPT, body, 0)
            pltpu.sync_copy(acc, d1p_hbm.at[sid])

        @pl.when(cid == 1)
        def _():
            def body(i, c):
                idx = jnp.full((16,), i, jnp.int32)
                d16 = plsc.load_gather(dbuf, [idx])
                plsc.addupdate_scatter(acc, [d16], jnp.ones((16,), jnp.float32),
                                       mask=lane0)
                return c
            lax.fori_loop(0, EPT, body, 0)
            pltpu.sync_copy(acc, d2p_hbm.at[sid])

    return deg_kernel(dst, ev)


def _sc_aggregate_pass(yflat, edata, weighted, k):
    mesh = plsc.VectorSubcoreMesh(core_axis_name="c", subcore_axis_name="s")
    NBLK = EPT // B
    NBUF = 4
    RPH = NPAD // NS

    @functools.partial(
        pl.kernel,
        out_type=jax.ShapeDtypeStruct((NC * NPAD, CW), jnp.float32),
        mesh=mesh,
        compiler_params=pltpu.CompilerParams(needs_layout_passes=False),
        scratch_types=(
            [pltpu.VMEM((4, B), jnp.int32) for _ in range(NBUF)] +
            [pltpu.VMEM((B, CW), jnp.float32) for _ in range(NBUF)] +
            [pltpu.VMEM_SHARED((NPAD, CW), jnp.float32)] +
            [pltpu.SemaphoreType.DMA for _ in range(2 * NBUF)]
        ),
    )
    def agg_kernel(y_hbm, edata_hbm, out_hbm, *scratch):
        ibs = scratch[:NBUF]
        bufs = scratch[NBUF:2 * NBUF]
        acc = scratch[2 * NBUF]
        gsems = scratch[2 * NBUF + 1:2 * NBUF + 1 + NBUF]
        ssems = scratch[2 * NBUF + 1 + NBUF:]
        cid = lax.axis_index("c")
        sid = lax.axis_index("s")
        chunk = cid * (NCH // NC) + k
        cbase = chunk * NPAD

        def load_start_g(p, j):
            ib = ibs[p]
            pltpu.sync_copy(edata_hbm.at[sid, j], ib)
            for u in range(B // 16):
                sl = pl.ds(u * 16, 16)
                ib[3, sl] = ib[0, sl] + cbase
            pltpu.async_copy(y_hbm.at[ib.at[3]], bufs[p], gsems[p])

        def wait_g(p):
            pltpu.make_async_copy(y_hbm.at[pl.ds(0, B)], bufs[p], gsems[p]).wait()

        def wait_s(p):
            pltpu.make_async_copy(bufs[p], acc.at[pl.ds(0, B)], ssems[p]).wait()

        def scale(p):
            if weighted:
                buf = bufs[p]
                ev = ibs[p].at[2]

                @plsc.parallel_loop(0, B, 1, unroll=8)
                def _(i):
                    w = plsc.bitcast(
                        plsc.load_gather(ev, [jnp.full((16,), i, jnp.int32)]),
                        jnp.float32)
                    for q in range(CW // 16):
                        sl = pl.ds(q * 16, 16)
                        buf[i, sl] = buf[i, sl] * w

        def process(p):
            wait_g(p)
            scale(p)
            pltpu.async_copy(bufs[p], acc.at[ibs[p].at[1]], ssems[p], add=True)

        pltpu.sync_copy(y_hbm.at[pl.ds(cbase + sid * RPH, RPH)],
                        acc.at[pl.ds(sid * RPH, RPH)])
        plsc.subcore_barrier()

        for p in range(NBUF):
            load_start_g(p, p)

        def body(m, c):
            for p in range(NBUF):
                process(p)
            for p in range(NBUF):
                wait_s(p)
                load_start_g(p, NBUF * m + NBUF + p)
            return c
        lax.fori_loop(0, NBLK // NBUF - 1, body, 0)
        for p in range(NBUF):
            process(p)
        for t in range(NBLK % NBUF):
            wait_s(t)
            load_start_g(t, (NBLK // NBUF) * NBUF + t)
            process(t)
        for p in range(NBUF):
            wait_s(p)

        plsc.subcore_barrier()
        pltpu.sync_copy(acc.at[pl.ds(sid * RPH, RPH)],
                        out_hbm.at[pl.ds(cid * NPAD + sid * RPH, RPH)])

    return agg_kernel(yflat, edata)


def _sc_aggregate(yflat, edata, weighted):
    return tuple(_sc_aggregate_pass(yflat, edata, weighted, k)
                 .reshape(NC, NPAD, CW)
                 for k in range(NCH // NC))


def _tc_stage1(xpad, W1, d1p, d2p):
    def body(x_ref, w1_ref, d1p_ref, d2p_ref, y1_ref, dinv_ref):
        xw = jnp.dot(x_ref[...], w1_ref[...], preferred_element_type=jnp.float32)
        dinv1 = lax.rsqrt(jnp.sum(d1p_ref[...], axis=0) + 1.0)
        dinv2 = lax.rsqrt(jnp.sum(d2p_ref[...], axis=0) + 1.0)
        y = xw * dinv1[:, None]
        for c in range(NCH):
            y1_ref[c] = y[:, c * CW:(c + 1) * CW]
        dinv_ref[0] = dinv1
        dinv_ref[1] = dinv2

    return pl.pallas_call(
        body,
        grid=(NB,),
        in_specs=[
            pl.BlockSpec((1024, D_IN), lambda i: (i, 0)),
            pl.BlockSpec((D_IN, D_H), lambda i: (0, 0)),
            pl.BlockSpec((NS, 1024), lambda i: (0, i)),
            pl.BlockSpec((NS, 1024), lambda i: (0, i)),
        ],
        out_specs=[
            pl.BlockSpec((NCH, 1024, CW), lambda i: (0, i, 0)),
            pl.BlockSpec((2, 1024), lambda i: (0, i)),
        ],
        out_shape=[
            jax.ShapeDtypeStruct((NCH, NPAD, CW), jnp.float32),
            jax.ShapeDtypeStruct((2, NPAD), jnp.float32),
        ],
    )(xpad, W1, d1p, d2p)


def _tc_stage2(g1a, g1b, dinvs, b1, W2):
    def body(ga_ref, gb_ref, dinv_ref, b1_ref, w2_ref, y2_ref):
        s = jnp.concatenate([ga_ref[0], gb_ref[0], ga_ref[1], gb_ref[1]],
                            axis=1)
        h = jnp.maximum(s * dinv_ref[0][:, None] + b1_ref[...][None, :], 0.0)
        hw = jnp.dot(h, w2_ref[...], preferred_element_type=jnp.float32)
        y2 = hw * dinv_ref[1][:, None]
        for c in range(NCH):
            y2_ref[c] = y2[:, c * CW:(c + 1) * CW]

    return pl.pallas_call(
        body,
        grid=(NB,),
        in_specs=[
            pl.BlockSpec((NC, 1024, CW), lambda i: (0, i, 0)),
            pl.BlockSpec((NC, 1024, CW), lambda i: (0, i, 0)),
            pl.BlockSpec((2, 1024), lambda i: (0, i)),
            pl.BlockSpec((D_H,), lambda i: (0,)),
            pl.BlockSpec((D_H, D_H), lambda i: (0, 0)),
        ],
        out_specs=pl.BlockSpec((NCH, 1024, CW), lambda i: (0, i, 0)),
        out_shape=jax.ShapeDtypeStruct((NCH, NPAD, CW), jnp.float32),
    )(g1a, g1b, dinvs, b1, W2)


def _tc_stage3(g2a, g2b, dinvs, b2, Wf, bf):
    def body(ga_ref, gb_ref, dinv_ref, b2_ref, wf_ref, bf_ref, out_ref,
             acc_ref):
        i = pl.program_id(0)
        s = jnp.concatenate([ga_ref[0], gb_ref[0], ga_ref[1], gb_ref[1]],
                            axis=1)
        h2 = jnp.maximum(s * dinv_ref[1][:, None] + b2_ref[...][None, :], 0.0)
        row = lax.broadcasted_iota(jnp.int32, (1024, 1), 0) + i * 1024
        h2 = jnp.where(row < N, h2, 0.0)
        psum = jnp.sum(h2, axis=0, keepdims=True)

        @pl.when(i == 0)
        def _():
            acc_ref[...] = psum

        @pl.when(i > 0)
        def _():
            acc_ref[...] += psum

        @pl.when(i == NB - 1)
        def _():
            pooled = acc_ref[...] * (1.0 / N)
            t = jnp.dot(pooled, wf_ref[...], preferred_element_type=jnp.float32)
            t = jnp.maximum(t + bf_ref[...][None, :], 0.0)
            m = jnp.max(t, axis=-1, keepdims=True)
            e = jnp.exp(t - m)
            out_ref[...] = e / jnp.sum(e, axis=-1, keepdims=True)

    return pl.pallas_call(
        body,
        grid=(NB,),
        in_specs=[
            pl.BlockSpec((NC, 1024, CW), lambda i: (0, i, 0)),
            pl.BlockSpec((NC, 1024, CW), lambda i: (0, i, 0)),
            pl.BlockSpec((2, 1024), lambda i: (0, i)),
            pl.BlockSpec((D_H,), lambda i: (0,)),
            pl.BlockSpec((D_H, D_H), lambda i: (0, 0)),
            pl.BlockSpec((D_H,), lambda i: (0,)),
        ],
        out_specs=pl.BlockSpec((1, D_H), lambda i: (0, 0)),
        out_shape=jax.ShapeDtypeStruct((1, D_H), jnp.float32),
        scratch_shapes=[pltpu.VMEM((1, D_H), jnp.float32)],
    )(g2a, g2b, dinvs, b2, Wf, bf)


def kernel(x, edge_index, edge_values, batch, W1, b1, W2, b2, Wf, bf):
    src = edge_index[0]
    dst = edge_index[1]
    xpad = jnp.pad(x, ((0, NPAD - N), (0, 0)))
    nblk = EPT // B
    evbits = lax.bitcast_convert_type(edge_values, jnp.int32)
    edata = jnp.stack([src.reshape(NS, nblk, B), dst.reshape(NS, nblk, B),
                       evbits.reshape(NS, nblk, B),
                       jnp.zeros((NS, nblk, B), jnp.int32)], axis=2)

    d1p, d2p = _sc_degrees(dst, edge_values)
    y1, dinvs = _tc_stage1(xpad, W1, d1p, d2p)
    g1a, g1b = _sc_aggregate(y1.reshape(NCH * NPAD, CW), edata, weighted=True)
    y2 = _tc_stage2(g1a, g1b, dinvs, b1, W2)
    g2a, g2b = _sc_aggregate(y2.reshape(NCH * NPAD, CW), edata, weighted=False)
    return _tc_stage3(g2a, g2b, dinvs, b2, Wf, bf)


# speedup vs baseline: 2.8019x; 1.0319x over previous
import functools

import jax
import jax.numpy as jnp
from jax import lax
from jax.experimental import pallas as pl
from jax.experimental.pallas import tpu as pltpu
from jax.experimental.pallas import tpu_sc as plsc

N = 10000
NPAD = 10240
E = 160000
D_IN = 256
D_H = 512
NCH = 4
CW = 128
NC = 2
NS = 16
EPT = E // NS
B = 80
NPH = NPAD // 2
RPT = NPAD // NS
NB = NPAD // 1024


def _sc_degrees(dst, ev):
    mesh = plsc.VectorSubcoreMesh(core_axis_name="c", subcore_axis_name="s")

    @functools.partial(
        pl.kernel,
        out_type=(jax.ShapeDtypeStruct((NS, NPAD), jnp.float32),
                  jax.ShapeDtypeStruct((NS, NPAD), jnp.float32)),
        mesh=mesh,
        compiler_params=pltpu.CompilerParams(needs_layout_passes=False),
        scratch_types=[
            pltpu.VMEM((EPT,), jnp.int32),
            pltpu.VMEM((EPT,), jnp.float32),
            pltpu.VMEM((16, NPH), jnp.float32),
            pltpu.VMEM((NPH,), jnp.float32),
        ],
    )
    def deg_kernel(dst_hbm, ev_hbm, d1p_hbm, d2p_hbm, dbuf, evbuf, acc2, red):
        cid = lax.axis_index("c")
        sid = lax.axis_index("s")
        base = sid * EPT
        pltpu.sync_copy(dst_hbm.at[pl.ds(base, EPT)], dbuf)

        @pl.when(cid == 0)
        def _():
            pltpu.sync_copy(ev_hbm.at[pl.ds(base, EPT)], evbuf)

        lanes = lax.iota(jnp.int32, 16)
        zero16 = jnp.zeros((16,), jnp.float32)
        one16 = jnp.ones((16,), jnp.float32)
        for r in range(2):
            lo = r * NPH

            @plsc.parallel_loop(0, 16 * (NPH // 16), 1, unroll=8)
            def _(t):
                acc2[t // (NPH // 16), pl.ds((t % (NPH // 16)) * 16, 16)] = zero16

            def scat(t, c2):
                for u in range(5):
                    sl = pl.ds((t * 5 + u) * 16, 16)
                    d16 = dbuf[sl]
                    v16 = jnp.where(cid == 0, evbuf[sl], one16)
                    m = (d16 >= lo) & (d16 < lo + NPH)
                    dc = jnp.where(m, d16 - lo, 0)
                    plsc.addupdate_scatter(acc2, [lanes, dc], v16, mask=m)
                return c2
            lax.fori_loop(0, EPT // 80, scat, 0)

            @plsc.parallel_loop(0, NPH // 16, 1, unroll=2)
            def _(g):
                sl = pl.ds(g * 16, 16)
                s = acc2[0, sl]
                for l in range(1, 16):
                    s = s + acc2[l, sl]
                red[sl] = s

            @pl.when(cid == 0)
            def _():
                pltpu.sync_copy(red, d1p_hbm.at[sid, pl.ds(lo, NPH)])

            @pl.when(cid == 1)
            def _():
                pltpu.sync_copy(red, d2p_hbm.at[sid, pl.ds(lo, NPH)])

    return deg_kernel(dst, ev)


def _sc_aggregate_pass(yflat, edata, weighted, k):
    mesh = plsc.VectorSubcoreMesh(core_axis_name="c", subcore_axis_name="s")
    NBLK = EPT // B
    NBUF = 4
    RPH = NPAD // NS

    @functools.partial(
        pl.kernel,
        out_type=jax.ShapeDtypeStruct((NC * NPAD, CW), jnp.float32),
        mesh=mesh,
        compiler_params=pltpu.CompilerParams(needs_layout_passes=False),
        scratch_types=(
            [pltpu.VMEM((4, B), jnp.int32) for _ in range(NBUF)] +
            [pltpu.VMEM((B, CW), jnp.float32) for _ in range(NBUF)] +
            [pltpu.VMEM_SHARED((NPAD, CW), jnp.float32)] +
            [pltpu.SemaphoreType.DMA for _ in range(2 * NBUF)]
        ),
    )
    def agg_kernel(y_hbm, edata_hbm, out_hbm, *scratch):
        ibs = scratch[:NBUF]
        bufs = scratch[NBUF:2 * NBUF]
        acc = scratch[2 * NBUF]
        gsems = scratch[2 * NBUF + 1:2 * NBUF + 1 + NBUF]
        ssems = scratch[2 * NBUF + 1 + NBUF:]
        cid = lax.axis_index("c")
        sid = lax.axis_index("s")
        chunk = cid * (NCH // NC) + k
        cbase = chunk * NPAD

        def load_start_g(p, j):
            ib = ibs[p]
            pltpu.sync_copy(edata_hbm.at[sid, j], ib)
            for u in range(B // 16):
                sl = pl.ds(u * 16, 16)
                ib[3, sl] = ib[0, sl] + cbase
            pltpu.async_copy(y_hbm.at[ib.at[3]], bufs[p], gsems[p])

        def wait_g(p):
            pltpu.make_async_copy(y_hbm.at[pl.ds(0, B)], bufs[p], gsems[p]).wait()

        def wait_s(p):
            pltpu.make_async_copy(bufs[p], acc.at[pl.ds(0, B)], ssems[p]).wait()

        def scale(p):
            if weighted:
                buf = bufs[p]
                ev = ibs[p].at[2]

                @plsc.parallel_loop(0, B, 1, unroll=8)
                def _(i):
                    w = plsc.bitcast(
                        plsc.load_gather(ev, [jnp.full((16,), i, jnp.int32)]),
                        jnp.float32)
                    for q in range(CW // 16):
                        sl = pl.ds(q * 16, 16)
                        buf[i, sl] = buf[i, sl] * w

        def process(p):
            wait_g(p)
            scale(p)
            pltpu.async_copy(bufs[p], acc.at[ibs[p].at[1]], ssems[p], add=True)

        pltpu.sync_copy(y_hbm.at[pl.ds(cbase + sid * RPH, RPH)],
                        acc.at[pl.ds(sid * RPH, RPH)])
        plsc.subcore_barrier()

        for p in range(NBUF):
            load_start_g(p, p)

        def body(m, c):
            for p in range(NBUF):
                process(p)
            for p in range(NBUF):
                wait_s(p)
                load_start_g(p, NBUF * m + NBUF + p)
            return c
        lax.fori_loop(0, NBLK // NBUF - 1, body, 0)
        for p in range(NBUF):
            process(p)
        for t in range(NBLK % NBUF):
            wait_s(t)
            load_start_g(t, (NBLK // NBUF) * NBUF + t)
            process(t)
        for p in range(NBUF):
            wait_s(p)

        plsc.subcore_barrier()
        pltpu.sync_copy(acc.at[pl.ds(sid * RPH, RPH)],
                        out_hbm.at[pl.ds(cid * NPAD + sid * RPH, RPH)])

    return agg_kernel(yflat, edata)


def _sc_aggregate(yflat, edata, weighted):
    return tuple(_sc_aggregate_pass(yflat, edata, weighted, k)
                 .reshape(NC, NPAD, CW)
                 for k in range(NCH // NC))


def _tc_stage1(xpad, W1, d1p, d2p):
    def body(x_ref, w1_ref, d1p_ref, d2p_ref, y1_ref, dinv_ref):
        xw = jnp.dot(x_ref[...], w1_ref[...], preferred_element_type=jnp.float32)
        dinv1 = lax.rsqrt(jnp.sum(d1p_ref[...], axis=0) + 1.0)
        dinv2 = lax.rsqrt(jnp.sum(d2p_ref[...], axis=0) + 1.0)
        y = xw * dinv1[:, None]
        for c in range(NCH):
            y1_ref[c] = y[:, c * CW:(c + 1) * CW]
        dinv_ref[0] = dinv1
        dinv_ref[1] = dinv2

    return pl.pallas_call(
        body,
        grid=(NB,),
        in_specs=[
            pl.BlockSpec((1024, D_IN), lambda i: (i, 0)),
            pl.BlockSpec((D_IN, D_H), lambda i: (0, 0)),
            pl.BlockSpec((NS, 1024), lambda i: (0, i)),
            pl.BlockSpec((NS, 1024), lambda i: (0, i)),
        ],
        out_specs=[
            pl.BlockSpec((NCH, 1024, CW), lambda i: (0, i, 0)),
            pl.BlockSpec((2, 1024), lambda i: (0, i)),
        ],
        out_shape=[
            jax.ShapeDtypeStruct((NCH, NPAD, CW), jnp.float32),
            jax.ShapeDtypeStruct((2, NPAD), jnp.float32),
        ],
    )(xpad, W1, d1p, d2p)


def _tc_stage2(g1a, g1b, dinvs, b1, W2):
    def body(ga_ref, gb_ref, dinv_ref, b1_ref, w2_ref, y2_ref):
        s = jnp.concatenate([ga_ref[0], gb_ref[0], ga_ref[1], gb_ref[1]],
                            axis=1)
        h = jnp.maximum(s * dinv_ref[0][:, None] + b1_ref[...][None, :], 0.0)
        hw = jnp.dot(h, w2_ref[...], preferred_element_type=jnp.float32)
        y2 = hw * dinv_ref[1][:, None]
        for c in range(NCH):
            y2_ref[c] = y2[:, c * CW:(c + 1) * CW]

    return pl.pallas_call(
        body,
        grid=(NB,),
        in_specs=[
            pl.BlockSpec((NC, 1024, CW), lambda i: (0, i, 0)),
            pl.BlockSpec((NC, 1024, CW), lambda i: (0, i, 0)),
            pl.BlockSpec((2, 1024), lambda i: (0, i)),
            pl.BlockSpec((D_H,), lambda i: (0,)),
            pl.BlockSpec((D_H, D_H), lambda i: (0, 0)),
        ],
        out_specs=pl.BlockSpec((NCH, 1024, CW), lambda i: (0, i, 0)),
        out_shape=jax.ShapeDtypeStruct((NCH, NPAD, CW), jnp.float32),
    )(g1a, g1b, dinvs, b1, W2)


def _tc_stage3(g2a, g2b, dinvs, b2, Wf, bf):
    def body(ga_ref, gb_ref, dinv_ref, b2_ref, wf_ref, bf_ref, out_ref,
             acc_ref):
        i = pl.program_id(0)
        s = jnp.concatenate([ga_ref[0], gb_ref[0], ga_ref[1], gb_ref[1]],
                            axis=1)
        h2 = jnp.maximum(s * dinv_ref[1][:, None] + b2_ref[...][None, :], 0.0)
        row = lax.broadcasted_iota(jnp.int32, (1024, 1), 0) + i * 1024
        h2 = jnp.where(row < N, h2, 0.0)
        psum = jnp.sum(h2, axis=0, keepdims=True)

        @pl.when(i == 0)
        def _():
            acc_ref[...] = psum

        @pl.when(i > 0)
        def _():
            acc_ref[...] += psum

        @pl.when(i == NB - 1)
        def _():
            pooled = acc_ref[...] * (1.0 / N)
            t = jnp.dot(pooled, wf_ref[...], preferred_element_type=jnp.float32)
            t = jnp.maximum(t + bf_ref[...][None, :], 0.0)
            m = jnp.max(t, axis=-1, keepdims=True)
            e = jnp.exp(t - m)
            out_ref[...] = e / jnp.sum(e, axis=-1, keepdims=True)

    return pl.pallas_call(
        body,
        grid=(NB,),
        in_specs=[
            pl.BlockSpec((NC, 1024, CW), lambda i: (0, i, 0)),
            pl.BlockSpec((NC, 1024, CW), lambda i: (0, i, 0)),
            pl.BlockSpec((2, 1024), lambda i: (0, i)),
            pl.BlockSpec((D_H,), lambda i: (0,)),
            pl.BlockSpec((D_H, D_H), lambda i: (0, 0)),
            pl.BlockSpec((D_H,), lambda i: (0,)),
        ],
        out_specs=pl.BlockSpec((1, D_H), lambda i: (0, 0)),
        out_shape=jax.ShapeDtypeStruct((1, D_H), jnp.float32),
        scratch_shapes=[pltpu.VMEM((1, D_H), jnp.float32)],
    )(g2a, g2b, dinvs, b2, Wf, bf)


def kernel(x, edge_index, edge_values, batch, W1, b1, W2, b2, Wf, bf):
    src = edge_index[0]
    dst = edge_index[1]
    xpad = jnp.pad(x, ((0, NPAD - N), (0, 0)))
    nblk = EPT // B
    evbits = lax.bitcast_convert_type(edge_values, jnp.int32)
    edata = jnp.stack([src.reshape(NS, nblk, B), dst.reshape(NS, nblk, B),
                       evbits.reshape(NS, nblk, B),
                       jnp.zeros((NS, nblk, B), jnp.int32)], axis=2)

    d1p, d2p = _sc_degrees(dst, edge_values)
    y1, dinvs = _tc_stage1(xpad, W1, d1p, d2p)
    g1a, g1b = _sc_aggregate(y1.reshape(NCH * NPAD, CW), edata, weighted=True)
    y2 = _tc_stage2(g1a, g1b, dinvs, b1, W2)
    g2a, g2b = _sc_aggregate(y2.reshape(NCH * NPAD, CW), edata, weighted=False)
    return _tc_stage3(g2a, g2b, dinvs, b2, Wf, bf)


# speedup vs baseline: 2.8111x; 1.0033x over previous
import functools

import jax
import jax.numpy as jnp
from jax import lax
from jax.experimental import pallas as pl
from jax.experimental.pallas import tpu as pltpu
from jax.experimental.pallas import tpu_sc as plsc

N = 10000
NPAD = 10240
E = 160000
D_IN = 256
D_H = 512
NCH = 4
CW = 128
NC = 2
NS = 16
EPT = E // NS
B = 80
NPH = NPAD // 2
RPT = NPAD // NS
NB = NPAD // 1024


def _sc_degrees(dst, ev):
    mesh = plsc.VectorSubcoreMesh(core_axis_name="c", subcore_axis_name="s")

    @functools.partial(
        pl.kernel,
        out_type=(jax.ShapeDtypeStruct((NS, NPAD), jnp.float32),
                  jax.ShapeDtypeStruct((NS, NPAD), jnp.float32)),
        mesh=mesh,
        compiler_params=pltpu.CompilerParams(needs_layout_passes=False),
        scratch_types=[
            pltpu.VMEM((EPT,), jnp.int32),
            pltpu.VMEM((EPT,), jnp.float32),
            pltpu.VMEM((16, NPH), jnp.float32),
            pltpu.VMEM((NPH,), jnp.float32),
        ],
    )
    def deg_kernel(dst_hbm, ev_hbm, d1p_hbm, d2p_hbm, dbuf, evbuf, acc2, red):
        cid = lax.axis_index("c")
        sid = lax.axis_index("s")
        base = sid * EPT
        pltpu.sync_copy(dst_hbm.at[pl.ds(base, EPT)], dbuf)

        @pl.when(cid == 0)
        def _():
            pltpu.sync_copy(ev_hbm.at[pl.ds(base, EPT)], evbuf)

        lanes = lax.iota(jnp.int32, 16)
        zero16 = jnp.zeros((16,), jnp.float32)
        one16 = jnp.ones((16,), jnp.float32)
        for r in range(2):
            lo = r * NPH

            @plsc.parallel_loop(0, 16 * (NPH // 16), 1, unroll=8)
            def _(t):
                acc2[t // (NPH // 16), pl.ds((t % (NPH // 16)) * 16, 16)] = zero16

            def scat(t, c2):
                for u in range(5):
                    sl = pl.ds((t * 5 + u) * 16, 16)
                    d16 = dbuf[sl]
                    v16 = jnp.where(cid == 0, evbuf[sl], one16)
                    m = (d16 >= lo) & (d16 < lo + NPH)
                    dc = jnp.where(m, d16 - lo, 0)
                    plsc.addupdate_scatter(acc2, [lanes, dc], v16, mask=m)
                return c2
            lax.fori_loop(0, EPT // 80, scat, 0)

            @plsc.parallel_loop(0, NPH // 16, 1, unroll=2)
            def _(g):
                sl = pl.ds(g * 16, 16)
                s = acc2[0, sl]
                for l in range(1, 16):
                    s = s + acc2[l, sl]
                red[sl] = s

            @pl.when(cid == 0)
            def _():
                pltpu.sync_copy(red, d1p_hbm.at[sid, pl.ds(lo, NPH)])

            @pl.when(cid == 1)
            def _():
                pltpu.sync_copy(red, d2p_hbm.at[sid, pl.ds(lo, NPH)])

    return deg_kernel(dst, ev)


def _sc_aggregate_pass(yflat, edata, weighted, k):
    mesh = plsc.VectorSubcoreMesh(core_axis_name="c", subcore_axis_name="s")
    NBLK = EPT // B
    NBUF = 4
    RPH = NPAD // NS

    @functools.partial(
        pl.kernel,
        out_type=jax.ShapeDtypeStruct((NC * NPAD, CW), jnp.float32),
        mesh=mesh,
        compiler_params=pltpu.CompilerParams(needs_layout_passes=False),
        scratch_types=(
            [pltpu.VMEM((4, B), jnp.int32) for _ in range(NBUF)] +
            [pltpu.VMEM((B, CW), jnp.float32) for _ in range(NBUF)] +
            [pltpu.VMEM_SHARED((NPAD, CW), jnp.float32)] +
            [pltpu.SemaphoreType.DMA for _ in range(2 * NBUF)]
        ),
    )
    def agg_kernel(y_hbm, edata_hbm, out_hbm, *scratch):
        ibs = scratch[:NBUF]
        bufs = scratch[NBUF:2 * NBUF]
        acc = scratch[2 * NBUF]
        gsems = scratch[2 * NBUF + 1:2 * NBUF + 1 + NBUF]
        ssems = scratch[2 * NBUF + 1 + NBUF:]
        cid = lax.axis_index("c")
        sid = lax.axis_index("s")
        chunk = cid * (NCH // NC) + k
        cbase = chunk * NPAD

        def load_start_g(p, j):
            ib = ibs[p]
            pltpu.sync_copy(edata_hbm.at[sid, j], ib)
            for u in range(B // 16):
                sl = pl.ds(u * 16, 16)
                ib[3, sl] = ib[0, sl] + cbase
            pltpu.async_copy(y_hbm.at[ib.at[3]], bufs[p], gsems[p])

        def wait_g(p):
            pltpu.make_async_copy(y_hbm.at[pl.ds(0, B)], bufs[p], gsems[p]).wait()

        def wait_s(p):
            pltpu.make_async_copy(bufs[p], acc.at[pl.ds(0, B)], ssems[p]).wait()

        def scale(p):
            if weighted:
                buf = bufs[p]
                ev = ibs[p].at[2]

                @plsc.parallel_loop(0, B, 1, unroll=8)
                def _(i):
                    w = plsc.bitcast(
                        plsc.load_gather(ev, [jnp.full((16,), i, jnp.int32)]),
                        jnp.float32)
                    for q in range(CW // 16):
                        sl = pl.ds(q * 16, 16)
                        buf[i, sl] = buf[i, sl] * w

        def process(p):
            wait_g(p)
            scale(p)
            pltpu.async_copy(bufs[p], acc.at[ibs[p].at[1]], ssems[p], add=True)

        for p in range(NBUF):
            load_start_g(p, p)

        pltpu.sync_copy(y_hbm.at[pl.ds(cbase + sid * RPH, RPH)],
                        acc.at[pl.ds(sid * RPH, RPH)])
        plsc.subcore_barrier()

        def body(m, c):
            for p in range(NBUF):
                process(p)
            for p in range(NBUF):
                wait_s(p)
                load_start_g(p, NBUF * m + NBUF + p)
            return c
        lax.fori_loop(0, NBLK // NBUF - 1, body, 0)
        for p in range(NBUF):
            process(p)
        for t in range(NBLK % NBUF):
            wait_s(t)
            load_start_g(t, (NBLK // NBUF) * NBUF + t)
            process(t)
        for p in range(NBUF):
            wait_s(p)

        plsc.subcore_barrier()
        pltpu.sync_copy(acc.at[pl.ds(sid * RPH, RPH)],
                        out_hbm.at[pl.ds(cid * NPAD + sid * RPH, RPH)])

    return agg_kernel(yflat, edata)


def _sc_aggregate(yflat, edata, weighted):
    return tuple(_sc_aggregate_pass(yflat, edata, weighted, k)
                 .reshape(NC, NPAD, CW)
                 for k in range(NCH // NC))


def _tc_stage1(xpad, W1, d1p, d2p):
    def body(x_ref, w1_ref, d1p_ref, d2p_ref, y1_ref, dinv_ref):
        xw = jnp.dot(x_ref[...], w1_ref[...], preferred_element_type=jnp.float32)
        dinv1 = lax.rsqrt(jnp.sum(d1p_ref[...], axis=0) + 1.0)
        dinv2 = lax.rsqrt(jnp.sum(d2p_ref[...], axis=0) + 1.0)
        y = xw * dinv1[:, None]
        for c in range(NCH):
            y1_ref[c] = y[:, c * CW:(c + 1) * CW]
        dinv_ref[0] = dinv1
        dinv_ref[1] = dinv2

    return pl.pallas_call(
        body,
        grid=(NB,),
        in_specs=[
            pl.BlockSpec((1024, D_IN), lambda i: (i, 0)),
            pl.BlockSpec((D_IN, D_H), lambda i: (0, 0)),
            pl.BlockSpec((NS, 1024), lambda i: (0, i)),
            pl.BlockSpec((NS, 1024), lambda i: (0, i)),
        ],
        out_specs=[
            pl.BlockSpec((NCH, 1024, CW), lambda i: (0, i, 0)),
            pl.BlockSpec((2, 1024), lambda i: (0, i)),
        ],
        out_shape=[
            jax.ShapeDtypeStruct((NCH, NPAD, CW), jnp.float32),
            jax.ShapeDtypeStruct((2, NPAD), jnp.float32),
        ],
    )(xpad, W1, d1p, d2p)


def _tc_stage2(g1a, g1b, dinvs, b1, W2):
    def body(ga_ref, gb_ref, dinv_ref, b1_ref, w2_ref, y2_ref):
        s = jnp.concatenate([ga_ref[0], gb_ref[0], ga_ref[1], gb_ref[1]],
                            axis=1)
        h = jnp.maximum(s * dinv_ref[0][:, None] + b1_ref[...][None, :], 0.0)
        hw = jnp.dot(h, w2_ref[...], preferred_element_type=jnp.float32)
        y2 = hw * dinv_ref[1][:, None]
        for c in range(NCH):
            y2_ref[c] = y2[:, c * CW:(c + 1) * CW]

    return pl.pallas_call(
        body,
        grid=(NB,),
        in_specs=[
            pl.BlockSpec((NC, 1024, CW), lambda i: (0, i, 0)),
            pl.BlockSpec((NC, 1024, CW), lambda i: (0, i, 0)),
            pl.BlockSpec((2, 1024), lambda i: (0, i)),
            pl.BlockSpec((D_H,), lambda i: (0,)),
            pl.BlockSpec((D_H, D_H), lambda i: (0, 0)),
        ],
        out_specs=pl.BlockSpec((NCH, 1024, CW), lambda i: (0, i, 0)),
        out_shape=jax.ShapeDtypeStruct((NCH, NPAD, CW), jnp.float32),
    )(g1a, g1b, dinvs, b1, W2)


def _tc_stage3(g2a, g2b, dinvs, b2, Wf, bf):
    def body(ga_ref, gb_ref, dinv_ref, b2_ref, wf_ref, bf_ref, out_ref,
             acc_ref):
        i = pl.program_id(0)
        s = jnp.concatenate([ga_ref[0], gb_ref[0], ga_ref[1], gb_ref[1]],
                            axis=1)
        h2 = jnp.maximum(s * dinv_ref[1][:, None] + b2_ref[...][None, :], 0.0)
        row = lax.broadcasted_iota(jnp.int32, (1024, 1), 0) + i * 1024
        h2 = jnp.where(row < N, h2, 0.0)
        psum = jnp.sum(h2, axis=0, keepdims=True)

        @pl.when(i == 0)
        def _():
            acc_ref[...] = psum

        @pl.when(i > 0)
        def _():
            acc_ref[...] += psum

        @pl.when(i == NB - 1)
        def _():
            pooled = acc_ref[...] * (1.0 / N)
            t = jnp.dot(pooled, wf_ref[...], preferred_element_type=jnp.float32)
            t = jnp.maximum(t + bf_ref[...][None, :], 0.0)
            m = jnp.max(t, axis=-1, keepdims=True)
            e = jnp.exp(t - m)
            out_ref[...] = e / jnp.sum(e, axis=-1, keepdims=True)

    return pl.pallas_call(
        body,
        grid=(NB,),
        in_specs=[
            pl.BlockSpec((NC, 1024, CW), lambda i: (0, i, 0)),
            pl.BlockSpec((NC, 1024, CW), lambda i: (0, i, 0)),
            pl.BlockSpec((2, 1024), lambda i: (0, i)),
            pl.BlockSpec((D_H,), lambda i: (0,)),
            pl.BlockSpec((D_H, D_H), lambda i: (0, 0)),
            pl.BlockSpec((D_H,), lambda i: (0,)),
        ],
        out_specs=pl.BlockSpec((1, D_H), lambda i: (0, 0)),
        out_shape=jax.ShapeDtypeStruct((1, D_H), jnp.float32),
        scratch_shapes=[pltpu.VMEM((1, D_H), jnp.float32)],
    )(g2a, g2b, dinvs, b2, Wf, bf)


def kernel(x, edge_index, edge_values, batch, W1, b1, W2, b2, Wf, bf):
    src = edge_index[0]
    dst = edge_index[1]
    xpad = jnp.pad(x, ((0, NPAD - N), (0, 0)))
    nblk = EPT // B
    evbits = lax.bitcast_convert_type(edge_values, jnp.int32)
    edata = jnp.stack([src.reshape(NS, nblk, B), dst.reshape(NS, nblk, B),
                       evbits.reshape(NS, nblk, B),
                       jnp.zeros((NS, nblk, B), jnp.int32)], axis=2)

    d1p, d2p = _sc_degrees(dst, edge_values)
    y1, dinvs = _tc_stage1(xpad, W1, d1p, d2p)
    g1a, g1b = _sc_aggregate(y1.reshape(NCH * NPAD, CW), edata, weighted=True)
    y2 = _tc_stage2(g1a, g1b, dinvs, b1, W2)
    g2a, g2b = _sc_aggregate(y2.reshape(NCH * NPAD, CW), edata, weighted=False)
    return _tc_stage3(g2a, g2b, dinvs, b2, Wf, bf)
